# baseline (speedup 1.0000x reference)
import functools

import jax
import jax.numpy as jnp
from jax import lax
from jax.experimental import pallas as pl
from jax.experimental.pallas import tpu as pltpu
from jax.experimental.pallas import tpu_sc as plsc

NUM_EMB = 1000000
B = 16384
L = 20
D = 64

W = 16384
NBLK = (NUM_EMB + W - 1) // W
TC_GRID = NBLK // 2 + 1
H = TC_GRID * W
MAX_TBLK = NBLK - 1

NUM_WORKERS = 32
ROWS_PER_W = B // NUM_WORKERS
CB = 32
NCHUNK = ROWS_PER_W // CB
G = CB * L // 128
LANES = 16


def _tc_pack_body(a_ref, b_ref, out_ref):
    out_ref[...] = jnp.concatenate(
        [jnp.transpose(a_ref[...]), jnp.transpose(b_ref[...])], axis=1
    )


def _tc_pack(tT):
    return pl.pallas_call(
        _tc_pack_body,
        grid=(TC_GRID,),
        in_specs=[
            pl.BlockSpec((D, W), lambda g: (0, g)),
            pl.BlockSpec((D, W), lambda g: (0, jnp.minimum(g + TC_GRID, MAX_TBLK))),
        ],
        out_specs=pl.BlockSpec((W, 2 * D), lambda g: (g, 0)),
        out_shape=jax.ShapeDtypeStruct((H, 2 * D), jnp.float32),
    )(tT, tT)


def _sc_body(table_hbm, qidx_hbm, out_hbm,
             idx0, idx1, rows0, rows1, out0, out1,
             semg0, semg1, semo0, semo1, semi0, semi1):
    c = lax.axis_index("c")
    s = lax.axis_index("s")
    wid = s * 2 + c
    cbase = wid * NCHUNK

    def stage_idx(ci, idx_v, semi):
        b0 = (cbase + ci) * CB
        pltpu.async_copy(qidx_hbm.at[pl.ds(b0 * L, CB * L)], idx_v, semi)

    def drain_idx(idx_v, semi):
        pltpu.make_async_copy(
            qidx_hbm.at[pl.ds(0, CB * L)], idx_v, semi
        ).wait()

    def fire_gathers(idx_v, rows_v, semg):
        for j in range(G):
            pltpu.async_copy(
                table_hbm.at[idx_v.at[pl.ds(j * 128, 128)]],
                rows_v.at[pl.ds(j * 128, 128)],
                semg,
            )

    def stage(ci, idx_v, rows_v, semg, semi):
        stage_idx(ci, idx_v, semi)
        drain_idx(idx_v, semi)
        fire_gathers(idx_v, rows_v, semg)

    def drain_gathers(rows_v, semg):
        for j in range(G):
            pltpu.make_async_copy(
                table_hbm.at[pl.ds(0, 128)],
                rows_v.at[pl.ds(j * 128, 128)],
                semg,
            ).wait()

    def drain_out(out_v, semo):
        pltpu.make_async_copy(
            out_hbm.at[pl.ds(0, CB)], out_v, semo
        ).wait()

    def compute(ci, rows_v, out_v, semo):
        def item(i, _):
            r = i * L
            for d in range(D // LANES):
                sl = pl.ds(d * LANES, LANES)
                acc = rows_v[r, sl]
                for l in range(1, L):
                    acc = acc + rows_v[r + l, sl]
                out_v[i, sl] = acc
            return 0

        lax.fori_loop(0, CB, item, 0)
        pltpu.async_copy(out_v, out_hbm.at[pl.ds((cbase + ci) * CB, CB)], semo)

    stage(0, idx0, rows0, semg0, semi0)
    stage(1, idx1, rows1, semg1, semi1)

    def pair(k, _):
        c0 = 2 * k
        c1 = 2 * k + 1

        @pl.when(k > 0)
        def _():
            drain_out(out0, semo0)

        drain_gathers(rows0, semg0)

        @pl.when(c0 + 2 < NCHUNK)
        def _():
            stage_idx(c0 + 2, idx0, semi0)

        compute(c0, rows0, out0, semo0)

        @pl.when(c0 + 2 < NCHUNK)
        def _():
            drain_idx(idx0, semi0)
            fire_gathers(idx0, rows0, semg0)

        @pl.when(k > 0)
        def _():
            drain_out(out1, semo1)

        drain_gathers(rows1, semg1)

        @pl.when(c1 + 2 < NCHUNK)
        def _():
            stage_idx(c1 + 2, idx1, semi1)

        compute(c1, rows1, out1, semo1)

        @pl.when(c1 + 2 < NCHUNK)
        def _():
            drain_idx(idx1, semi1)
            fire_gathers(idx1, rows1, semg1)

        return 0

    lax.fori_loop(0, NCHUNK // 2, pair, 0)
    drain_out(out0, semo0)
    drain_out(out1, semo1)


def _sc_lookup_sum(packed, qidx):
    mesh = plsc.VectorSubcoreMesh(core_axis_name="c", subcore_axis_name="s")
    f = functools.partial(
        pl.kernel,
        mesh=mesh,
        compiler_params=pltpu.CompilerParams(use_tc_tiling_on_sc=False),
        out_type=jax.ShapeDtypeStruct((B, D), jnp.float32),
        scratch_types=[
            pltpu.VMEM((CB * L,), jnp.int32),
            pltpu.VMEM((CB * L,), jnp.int32),
            pltpu.VMEM((CB * L, D), jnp.float32),
            pltpu.VMEM((CB * L, D), jnp.float32),
            pltpu.VMEM((CB, D), jnp.float32),
            pltpu.VMEM((CB, D), jnp.float32),
            pltpu.SemaphoreType.DMA,
            pltpu.SemaphoreType.DMA,
            pltpu.SemaphoreType.DMA,
            pltpu.SemaphoreType.DMA,
            pltpu.SemaphoreType.DMA,
            pltpu.SemaphoreType.DMA,
        ],
    )(_sc_body)
    return f(packed, qidx)


@jax.jit
def _run(table, query):
    packed = _tc_pack(table.T).reshape(2 * H, D)
    q = query.astype(jnp.int32)
    qj = jnp.where(q < H, 2 * q, 2 * (q - H) + 1)
    return _sc_lookup_sum(packed, qj.reshape(B * L))


def kernel(table, query):
    return _run(table, query)


# speedup vs baseline: 1.1205x; 1.1205x over previous
import functools

import jax
import jax.numpy as jnp
from jax import lax
from jax.experimental import pallas as pl
from jax.experimental.pallas import tpu as pltpu
from jax.experimental.pallas import tpu_sc as plsc

NUM_EMB = 1000000
B = 16384
L = 20
D = 64

W = 16384
NBLK = (NUM_EMB + W - 1) // W
TC_GRID = NBLK // 2 + 1
H = TC_GRID * W
MAX_TBLK = NBLK - 1

NUM_WORKERS = 32
ROWS_PER_W = B // NUM_WORKERS
CB = 32
NCHUNK = ROWS_PER_W // CB
G = CB * L // 128
LANES = 16


def _tc_pack_body(a_ref, b_ref, out_ref):
    out_ref[...] = jnp.concatenate(
        [jnp.transpose(a_ref[...]), jnp.transpose(b_ref[...])], axis=1
    )


def _tc_pack(tT):
    return pl.pallas_call(
        _tc_pack_body,
        grid=(TC_GRID,),
        in_specs=[
            pl.BlockSpec((D, W), lambda g: (0, g)),
            pl.BlockSpec((D, W), lambda g: (0, jnp.minimum(g + TC_GRID, MAX_TBLK))),
        ],
        out_specs=pl.BlockSpec((W, 2 * D), lambda g: (g, 0)),
        out_shape=jax.ShapeDtypeStruct((H, 2 * D), jnp.float32),
    )(tT, tT)


def _sc_body(table_hbm, qidx_hbm, out_hbm,
             idx0, idx1, rows0, rows1, out0, out1,
             semg0, semg1, semo0, semo1, semi0, semi1):
    c = lax.axis_index("c")
    s = lax.axis_index("s")
    wid = s * 2 + c
    cbase = wid * NCHUNK

    def stage_idx(ci, idx_v, semi):
        b0 = (cbase + ci) * CB
        for l in range(L):
            pltpu.async_copy(
                qidx_hbm.at[pl.ds(l * B + b0, CB)],
                idx_v.at[pl.ds(l * CB, CB)],
                semi,
            )

    def drain_idx(idx_v, semi):
        pltpu.make_async_copy(
            qidx_hbm.at[pl.ds(0, CB * L)], idx_v, semi
        ).wait()

    def fire_gathers(idx_v, rows_v, semg):
        for j in range(G):
            pltpu.async_copy(
                table_hbm.at[idx_v.at[pl.ds(j * 128, 128)]],
                rows_v.at[pl.ds(j * 128, 128)],
                semg,
            )

    def stage(ci, idx_v, rows_v, semg, semi):
        stage_idx(ci, idx_v, semi)
        drain_idx(idx_v, semi)
        fire_gathers(idx_v, rows_v, semg)

    def drain_gathers(rows_v, semg):
        for j in range(G):
            pltpu.make_async_copy(
                table_hbm.at[pl.ds(0, 128)],
                rows_v.at[pl.ds(j * 128, 128)],
                semg,
            ).wait()

    def drain_out(out_v, semo):
        pltpu.make_async_copy(
            out_hbm.at[pl.ds(0, CB)], out_v, semo
        ).wait()

    def compute(ci, rows_v, out_v, semo):
        def item(i, _):
            for d in range(D // LANES):
                sl = pl.ds(d * LANES, LANES)
                acc = rows_v[i, sl]
                for l in range(1, L):
                    acc = acc + rows_v[l * CB + i, sl]
                out_v[i, sl] = acc
            return 0

        lax.fori_loop(0, CB, item, 0)
        pltpu.async_copy(out_v, out_hbm.at[pl.ds((cbase + ci) * CB, CB)], semo)

    stage(0, idx0, rows0, semg0, semi0)
    stage(1, idx1, rows1, semg1, semi1)

    def pair(k, _):
        c0 = 2 * k
        c1 = 2 * k + 1

        @pl.when(k > 0)
        def _():
            drain_out(out0, semo0)

        drain_gathers(rows0, semg0)

        @pl.when(c0 + 2 < NCHUNK)
        def _():
            stage_idx(c0 + 2, idx0, semi0)

        compute(c0, rows0, out0, semo0)

        @pl.when(c0 + 2 < NCHUNK)
        def _():
            drain_idx(idx0, semi0)
            fire_gathers(idx0, rows0, semg0)

        @pl.when(k > 0)
        def _():
            drain_out(out1, semo1)

        drain_gathers(rows1, semg1)

        @pl.when(c1 + 2 < NCHUNK)
        def _():
            stage_idx(c1 + 2, idx1, semi1)

        compute(c1, rows1, out1, semo1)

        @pl.when(c1 + 2 < NCHUNK)
        def _():
            drain_idx(idx1, semi1)
            fire_gathers(idx1, rows1, semg1)

        return 0

    lax.fori_loop(0, NCHUNK // 2, pair, 0)
    drain_out(out0, semo0)
    drain_out(out1, semo1)


def _sc_lookup_sum(packed, qidx):
    mesh = plsc.VectorSubcoreMesh(core_axis_name="c", subcore_axis_name="s")
    f = functools.partial(
        pl.kernel,
        mesh=mesh,
        compiler_params=pltpu.CompilerParams(use_tc_tiling_on_sc=False),
        out_type=jax.ShapeDtypeStruct((B, D), jnp.float32),
        scratch_types=[
            pltpu.VMEM((CB * L,), jnp.int32),
            pltpu.VMEM((CB * L,), jnp.int32),
            pltpu.VMEM((CB * L, D), jnp.float32),
            pltpu.VMEM((CB * L, D), jnp.float32),
            pltpu.VMEM((CB, D), jnp.float32),
            pltpu.VMEM((CB, D), jnp.float32),
            pltpu.SemaphoreType.DMA,
            pltpu.SemaphoreType.DMA,
            pltpu.SemaphoreType.DMA,
            pltpu.SemaphoreType.DMA,
            pltpu.SemaphoreType.DMA,
            pltpu.SemaphoreType.DMA,
        ],
    )(_sc_body)
    return f(packed, qidx)


@jax.jit
def _run(table, query):
    packed = _tc_pack(table.T).reshape(2 * H, D)
    q = query.T.astype(jnp.int32)
    qj = jnp.where(q < H, 2 * q, 2 * (q - H) + 1)
    return _sc_lookup_sum(packed, qj.reshape(L * B))


def kernel(table, query):
    return _run(table, query)
